# Initial kernel scaffold; baseline (speedup 1.0000x reference)
import jax
import jax.numpy as jnp
from jax.experimental import pallas as pl


def kernel(nodes, to_neighs, embed_table):
    raise NotImplementedError("write your pallas kernel here")


# speedup vs baseline: 5.2362x; 5.2362x over previous
import functools

import jax
import jax.numpy as jnp
from jax import lax
from jax.experimental import pallas as pl
from jax.experimental.pallas import tpu as pltpu
from jax.experimental.pallas import tpu_sc as plsc

B = 16384
S = 32
D = 128
LANES = 16
VPR = D // LANES

NC = 2
NS = 16
NW = NC * NS
B_PER_W = B // NW

C = 4
ROWS = C * S
N_CHUNK = B_PER_W // C

_SCALE = 1.0 / S


def _make_kernel():
    mesh = plsc.VectorSubcoreMesh(core_axis_name="c", subcore_axis_name="s")

    @functools.partial(
        pl.kernel,
        mesh=mesh,
        out_type=jax.ShapeDtypeStruct((B, D), jnp.float32),
        scratch_types=[
            pltpu.VMEM((ROWS,), jnp.int32),
            pltpu.VMEM((ROWS, D), jnp.float32),
            pltpu.VMEM((C, D), jnp.float32),
            pltpu.SemaphoreType.DMA,
        ],
    )
    def agg(neighs_hbm, table_hbm, out_hbm, idx_v, rows_v, res_v, sem):
        wid = lax.axis_index("s") * NC + lax.axis_index("c")
        w_base = wid * B_PER_W

        def chunk_body(g, _):
            row0 = w_base + g * C
            pltpu.sync_copy(neighs_hbm.at[pl.ds(row0 * S, ROWS)], idx_v)
            pltpu.async_copy(table_hbm.at[idx_v], rows_v, sem).wait()

            for c in range(C):
                def s_body(s, acc):
                    r = c * S + s
                    return tuple(
                        acc[j] + rows_v[r, pl.ds(j * LANES, LANES)]
                        for j in range(VPR)
                    )

                init = tuple(
                    jnp.zeros((LANES,), jnp.float32) for _ in range(VPR)
                )
                acc = lax.fori_loop(0, S, s_body, init)
                for j in range(VPR):
                    res_v[c, pl.ds(j * LANES, LANES)] = acc[j] * _SCALE

            pltpu.sync_copy(res_v, out_hbm.at[pl.ds(row0, C)])
            return 0

        lax.fori_loop(0, N_CHUNK, chunk_body, 0)

    return agg


_agg = _make_kernel()


def kernel(nodes, to_neighs, embed_table):
    del nodes
    neighs_flat = to_neighs.astype(jnp.int32).reshape(B * S)
    return _agg(neighs_flat, embed_table)
